# baseline (speedup 1.0000x reference)
import dataclasses

import jax
import jax.numpy as jnp
import numpy as np
from jax import lax
from jax.experimental import pallas as pl
from jax.experimental.pallas import tpu as pltpu
from jax.experimental.pallas import tpu_sc as plsc

ROWS = 128
N = 32768
L = 16
NUM_WORKERS = 32
ROWS_PER_WORKER = ROWS // NUM_WORKERS

DIGIT_BITS = (11, 11, 10)
NBINS = 2048

_MIN_I32 = np.int32(-(2**31))


def _digit(k, p):
    shift = sum(DIGIT_BITS[:p])
    mask = (1 << DIGIT_BITS[p]) - 1
    d = k if shift == 0 else lax.shift_right_logical(k, shift)
    return d & mask


def _sort_body(x_hbm, vals_hbm, idx_hbm, keys, pa, pb, h0, h1, h2, sem):
    wid = lax.axis_index("s") * 2 + lax.axis_index("c")

    @pl.loop(0, ROWS_PER_WORKER)
    def _row(r):
        row = wid * ROWS_PER_WORKER + r
        pltpu.async_copy(x_hbm.at[row], keys, sem).wait()

        hists = (h0, h1, h2)

        @pl.loop(0, NBINS, step=L)
        def _zero(h):
            z = jnp.zeros((L,), jnp.int32)
            h0[pl.ds(h, L)] = z
            h1[pl.ds(h, L)] = z

        @pl.loop(0, NBINS // 2, step=L)
        def _zero2(h):
            h2[pl.ds(h, L)] = jnp.zeros((L,), jnp.int32)

        def _flip_one(j):
            v = keys[pl.ds(j, L)]
            m = lax.shift_right_arithmetic(v, 31) | _MIN_I32
            f = v ^ m
            keys[pl.ds(j, L)] = f
            return _digit(f, 0), _digit(f, 1), _digit(f, 2)

        def _hist_upd(digs):
            cls = [plsc.scan_count(d) for d in digs]
            for hr, d, (cnt, last) in zip(hists, digs, cls):
                plsc.addupdate_scatter(hr, [d], cnt, mask=last)

        digs0 = _flip_one(0)

        @pl.loop(0, N - L, step=L, init_carry=digs0)
        def _flip(j, digs):
            c0, l0 = plsc.scan_count(digs[0])
            c1, l1 = plsc.scan_count(digs[1])
            c2, l2 = plsc.scan_count(digs[2])
            nxt = _flip_one(j + L)
            plsc.addupdate_scatter(h0, [digs[0]], c0, mask=l0)
            plsc.addupdate_scatter(h1, [digs[1]], c1, mask=l1)
            plsc.addupdate_scatter(h2, [digs[2]], c2, mask=l2)
            return nxt

        _hist_upd(_flip)

        for p in range(3):
            src, dst = ((None, pa), (pa, pb), (pb, pa))[p]
            hist = hists[p]
            nb = NBINS if p < 2 else NBINS // 2

            @pl.loop(0, nb, step=L, init_carry=np.int32(0))
            def _scan(h, carry):
                v = hist[pl.ds(h, L)]
                c = plsc.cumsum(v)
                hist[pl.ds(h, L)] = c - v + carry
                return carry + jnp.sum(v)

            def _pay_dig(j):
                if p == 0:
                    pay = lax.iota(jnp.int32, L) + j
                    k = keys[pl.ds(j, L)]
                else:
                    pay = src[pl.ds(j, L)]
                    k = plsc.load_gather(keys, [pay])
                return pay, _digit(k, p)

            def _scatter(pay, dig, cnt, last):
                base = plsc.load_gather(hist, [dig])
                plsc.store_scatter(dst, [base + cnt - 1], pay)
                plsc.addupdate_scatter(hist, [dig], cnt, mask=last)

            payb, digb = _pay_dig(0)
            cntb, lastb = plsc.scan_count(digb)
            pdc = _pay_dig(L)

            @pl.loop(0, N - 2 * L, step=L,
                     init_carry=(pdc, (payb, digb, cntb, lastb)))
            def _perm(j, carry):
                (payc, digc), (pay, dig, cnt, last) = carry
                cntc, lastc = plsc.scan_count(digc)
                pdd = _pay_dig(j + 2 * L)
                _scatter(pay, dig, cnt, last)
                return pdd, (payc, digc, cntc, lastc)

            (payc, digc), stb = _perm
            _scatter(*stb)
            cntc, lastc = plsc.scan_count(digc)
            _scatter(payc, digc, cntc, lastc)

        @pl.loop(0, N, step=L)
        def _vals(j):
            pay = pa[pl.ds(j, L)]
            k = plsc.load_gather(keys, [pay])
            m = (~lax.shift_right_arithmetic(k, 31)) | _MIN_I32
            pb[pl.ds(j, L)] = k ^ m

        pltpu.async_copy(pb, vals_hbm.at[row], sem).wait()
        pltpu.async_copy(pa, idx_hbm.at[row], sem).wait()


@jax.jit
def kernel(x):
    xi = lax.bitcast_convert_type(x, jnp.int32)
    mesh = plsc.VectorSubcoreMesh(core_axis_name="c", subcore_axis_name="s")
    cp = pltpu.CompilerParams()
    if "needs_layout_passes" in pltpu.CompilerParams.__dataclass_fields__:
        cp = dataclasses.replace(cp, needs_layout_passes=False)
    f = pl.kernel(
        _sort_body,
        out_type=(
            jax.ShapeDtypeStruct((ROWS, N), jnp.int32),
            jax.ShapeDtypeStruct((ROWS, N), jnp.int32),
        ),
        mesh=mesh,
        scratch_types=[
            pltpu.VMEM((N,), jnp.int32),
            pltpu.VMEM((N,), jnp.int32),
            pltpu.VMEM((N,), jnp.int32),
            pltpu.VMEM((NBINS,), jnp.int32),
            pltpu.VMEM((NBINS,), jnp.int32),
            pltpu.VMEM((NBINS // 2,), jnp.int32),
            pltpu.SemaphoreType.DMA,
        ],
        compiler_params=cp,
    )
    vals_i, idx = f(xi)
    return lax.bitcast_convert_type(vals_i, jnp.float32), idx


# speedup vs baseline: 1.0358x; 1.0358x over previous
import dataclasses

import jax
import jax.numpy as jnp
import numpy as np
from jax import lax
from jax.experimental import pallas as pl
from jax.experimental.pallas import tpu as pltpu
from jax.experimental.pallas import tpu_sc as plsc

ROWS = 128
N = 32768
L = 16
NUM_WORKERS = 32
ROWS_PER_WORKER = ROWS // NUM_WORKERS

DIGIT_BITS = (11, 11, 10)
NBINS = 2048

_MIN_I32 = np.int32(-(2**31))


def _flip(v):
    return v ^ (lax.shift_right_arithmetic(v, 31) | _MIN_I32)


def _digit(f, p):
    shift = sum(DIGIT_BITS[:p])
    mask = (1 << DIGIT_BITS[p]) - 1
    d = f if shift == 0 else lax.shift_right_logical(f, shift)
    return d & mask


def _sort_body(x_hbm, vals_hbm, idx_hbm, keys, pa, pb, h0, h1, h2, sem):
    wid = lax.axis_index("s") * 2 + lax.axis_index("c")
    hists = (h0, h1, h2)

    @pl.loop(0, ROWS_PER_WORKER)
    def _row(r):
        row = wid * ROWS_PER_WORKER + r
        pltpu.async_copy(x_hbm.at[row], keys, sem).wait()

        @pl.loop(0, NBINS, step=L)
        def _zero(h):
            z = jnp.zeros((L,), jnp.int32)
            h0[pl.ds(h, L)] = z
            h1[pl.ds(h, L)] = z

        @pl.loop(0, NBINS // 2, step=L)
        def _zero2(h):
            h2[pl.ds(h, L)] = jnp.zeros((L,), jnp.int32)

        def _digs(j):
            f = _flip(keys[pl.ds(j, L)])
            return _digit(f, 0), _digit(f, 1), _digit(f, 2)

        def _hist_upd(digs, cls):
            for hr, d, (cnt, last) in zip(hists, digs, cls):
                plsc.addupdate_scatter(hr, [d], cnt, mask=last)

        digs0 = _digs(0)

        @pl.loop(0, N - L, step=L, init_carry=digs0)
        def _hist(j, digs):
            cls = [plsc.scan_count(d) for d in digs]
            nxt = _digs(j + L)
            _hist_upd(digs, cls)
            return nxt

        _hist_upd(_hist, [plsc.scan_count(d) for d in _hist])

        for p in range(3):
            src, dst = ((None, pa), (pa, pb), (pb, pa))[p]
            hist = hists[p]
            nb = NBINS if p < 2 else NBINS // 2

            @pl.loop(0, nb, step=L, init_carry=np.int32(0))
            def _scan(h, carry):
                v = hist[pl.ds(h, L)]
                c = plsc.cumsum(v)
                hist[pl.ds(h, L)] = c - v + carry
                return carry + jnp.sum(v)

            def _sval_dig(j):
                if p == 0:
                    pay = lax.iota(jnp.int32, L) + j
                    f = _flip(keys[pl.ds(j, L)])
                    return pay | ((_digit(f, 1)) << 15), _digit(f, 0)
                packed = src[pl.ds(j, L)]
                pay = packed & 0x7FFF
                dig = lax.shift_right_logical(packed, 15)
                if p == 1:
                    f = _flip(plsc.load_gather(keys, [pay]))
                    return pay | (_digit(f, 2) << 15), dig
                return pay, dig

            def _scatter(sval, dig, cnt, last):
                base = plsc.load_gather(hist, [dig])
                plsc.store_scatter(dst, [base + cnt - 1], sval)
                plsc.addupdate_scatter(hist, [dig], cnt, mask=last)

            sd0 = _sval_dig(0)

            @pl.loop(0, N - L, step=L, init_carry=sd0)
            def _perm(j, carry):
                sval, dig = carry
                cnt, last = plsc.scan_count(dig)
                nxt = _sval_dig(j + L)
                _scatter(sval, dig, cnt, last)
                return nxt

            svalf, digf = _perm
            cntf, lastf = plsc.scan_count(digf)
            _scatter(svalf, digf, cntf, lastf)

        @pl.loop(0, N, step=L)
        def _vals(j):
            pay = pa[pl.ds(j, L)]
            pb[pl.ds(j, L)] = plsc.load_gather(keys, [pay])

        pltpu.async_copy(pb, vals_hbm.at[row], sem).wait()
        pltpu.async_copy(pa, idx_hbm.at[row], sem).wait()


@jax.jit
def kernel(x):
    xi = lax.bitcast_convert_type(x, jnp.int32)
    mesh = plsc.VectorSubcoreMesh(core_axis_name="c", subcore_axis_name="s")
    cp = pltpu.CompilerParams()
    if "needs_layout_passes" in pltpu.CompilerParams.__dataclass_fields__:
        cp = dataclasses.replace(cp, needs_layout_passes=False)
    f = pl.kernel(
        _sort_body,
        out_type=(
            jax.ShapeDtypeStruct((ROWS, N), jnp.int32),
            jax.ShapeDtypeStruct((ROWS, N), jnp.int32),
        ),
        mesh=mesh,
        scratch_types=[
            pltpu.VMEM((N,), jnp.int32),
            pltpu.VMEM((N,), jnp.int32),
            pltpu.VMEM((N,), jnp.int32),
            pltpu.VMEM((NBINS,), jnp.int32),
            pltpu.VMEM((NBINS,), jnp.int32),
            pltpu.VMEM((NBINS // 2,), jnp.int32),
            pltpu.SemaphoreType.DMA,
        ],
        compiler_params=cp,
    )
    vals_i, idx = f(xi)
    return lax.bitcast_convert_type(vals_i, jnp.float32), idx


# speedup vs baseline: 1.1082x; 1.0699x over previous
import dataclasses

import jax
import jax.numpy as jnp
import numpy as np
from jax import lax
from jax.experimental import pallas as pl
from jax.experimental.pallas import tpu as pltpu
from jax.experimental.pallas import tpu_sc as plsc

ROWS = 128
N = 32768
L = 16
NUM_WORKERS = 32
ROWS_PER_WORKER = ROWS // NUM_WORKERS

DIGIT_BITS = (11, 11, 10)
NBINS = 2048

_MIN_I32 = np.int32(-(2**31))


def _flip(v):
    return v ^ (lax.shift_right_arithmetic(v, 31) | _MIN_I32)


def _digit(f, p):
    shift = sum(DIGIT_BITS[:p])
    mask = (1 << DIGIT_BITS[p]) - 1
    d = f if shift == 0 else lax.shift_right_logical(f, shift)
    return d & mask


def _sort_body(x_hbm, vals_hbm, idx_hbm, keys, pa, pb, h0, h1, h2,
               sem_in, sem_v, sem_i):
    wid = lax.axis_index("s") * 2 + lax.axis_index("c")
    hists = (h0, h1, h2)
    row0 = wid * ROWS_PER_WORKER

    pltpu.async_copy(x_hbm.at[row0], keys, sem_in)

    @pl.loop(0, ROWS_PER_WORKER)
    def _row(r):
        row = row0 + r

        @pl.loop(0, NBINS, step=L)
        def _zero(h):
            z = jnp.zeros((L,), jnp.int32)
            h0[pl.ds(h, L)] = z
            h1[pl.ds(h, L)] = z

        @pl.loop(0, NBINS // 2, step=L)
        def _zero2(h):
            h2[pl.ds(h, L)] = jnp.zeros((L,), jnp.int32)

        pltpu.make_async_copy(x_hbm.at[row], keys, sem_in).wait()

        def _digs(j):
            f = _flip(keys[pl.ds(j, L)])
            keys[pl.ds(j, L)] = f
            return _digit(f, 0), _digit(f, 1), _digit(f, 2)

        def _hist_upd(digs, cls):
            for hr, d, (cnt, last) in zip(hists, digs, cls):
                plsc.addupdate_scatter(hr, [d], cnt, mask=last)

        digs0 = _digs(0)

        @pl.loop(0, N - L, step=L, init_carry=digs0)
        def _hist(j, digs):
            cls = [plsc.scan_count(d) for d in digs]
            nxt = _digs(j + L)
            _hist_upd(digs, cls)
            return nxt

        _hist_upd(_hist, [plsc.scan_count(d) for d in _hist])

        for p in range(3):
            src, dst = ((None, pa), (pa, pb), (pb, pa))[p]
            hist = hists[p]
            nb = NBINS if p < 2 else NBINS // 2

            if p < 2:
                @pl.when(r > 0)
                def _drain():
                    prev = row - 1
                    if p == 0:
                        pltpu.make_async_copy(pa, idx_hbm.at[prev], sem_i).wait()
                    else:
                        pltpu.make_async_copy(pb, vals_hbm.at[prev], sem_v).wait()

            @pl.loop(0, nb, step=L, init_carry=np.int32(0))
            def _scan(h, carry):
                v = hist[pl.ds(h, L)]
                c = plsc.cumsum(v)
                hist[pl.ds(h, L)] = c - v + carry
                return carry + jnp.sum(v)

            def _sval_dig(j):
                if p == 0:
                    pay = lax.iota(jnp.int32, L) + j
                    f = keys[pl.ds(j, L)]
                    return pay | ((_digit(f, 1)) << 15), _digit(f, 0)
                packed = src[pl.ds(j, L)]
                pay = packed & 0x7FFF
                dig = lax.shift_right_logical(packed, 15)
                if p == 1:
                    f = plsc.load_gather(keys, [pay])
                    return pay | (_digit(f, 2) << 15), dig
                return pay, dig

            def _scatter(sval, dig, cnt, last):
                base = plsc.load_gather(hist, [dig])
                plsc.store_scatter(dst, [base + cnt - 1], sval)
                plsc.addupdate_scatter(hist, [dig], cnt, mask=last)

            sd0 = _sval_dig(0)

            @pl.loop(0, N - L, step=L, init_carry=sd0)
            def _perm(j, carry):
                sval, dig = carry
                cnt, last = plsc.scan_count(dig)
                nxt = _sval_dig(j + L)
                _scatter(sval, dig, cnt, last)
                return nxt

            svalf, digf = _perm
            cntf, lastf = plsc.scan_count(digf)
            _scatter(svalf, digf, cntf, lastf)

        @pl.loop(0, N, step=L)
        def _vals(j):
            pay = pa[pl.ds(j, L)]
            k = plsc.load_gather(keys, [pay])
            m = (~lax.shift_right_arithmetic(k, 31)) | _MIN_I32
            pb[pl.ds(j, L)] = k ^ m

        pltpu.async_copy(pb, vals_hbm.at[row], sem_v)
        pltpu.async_copy(pa, idx_hbm.at[row], sem_i)

        @pl.when(r < ROWS_PER_WORKER - 1)
        def _next_in():
            pltpu.async_copy(x_hbm.at[row + 1], keys, sem_in)

    last = row0 + ROWS_PER_WORKER - 1
    pltpu.make_async_copy(pb, vals_hbm.at[last], sem_v).wait()
    pltpu.make_async_copy(pa, idx_hbm.at[last], sem_i).wait()


@jax.jit
def kernel(x):
    xi = lax.bitcast_convert_type(x, jnp.int32)
    mesh = plsc.VectorSubcoreMesh(core_axis_name="c", subcore_axis_name="s")
    cp = pltpu.CompilerParams()
    if "needs_layout_passes" in pltpu.CompilerParams.__dataclass_fields__:
        cp = dataclasses.replace(cp, needs_layout_passes=False)
    f = pl.kernel(
        _sort_body,
        out_type=(
            jax.ShapeDtypeStruct((ROWS, N), jnp.int32),
            jax.ShapeDtypeStruct((ROWS, N), jnp.int32),
        ),
        mesh=mesh,
        scratch_types=[
            pltpu.VMEM((N,), jnp.int32),
            pltpu.VMEM((N,), jnp.int32),
            pltpu.VMEM((N,), jnp.int32),
            pltpu.VMEM((NBINS,), jnp.int32),
            pltpu.VMEM((NBINS,), jnp.int32),
            pltpu.VMEM((NBINS // 2,), jnp.int32),
            pltpu.SemaphoreType.DMA,
            pltpu.SemaphoreType.DMA,
            pltpu.SemaphoreType.DMA,
        ],
        compiler_params=cp,
    )
    vals_i, idx = f(xi)
    return lax.bitcast_convert_type(vals_i, jnp.float32), idx
